# baseline (speedup 1.0000x reference)
import functools

import jax
import jax.numpy as jnp
from jax import lax
from jax.experimental import pallas as pl
from jax.experimental.pallas import tpu as pltpu
from jax.experimental.pallas import tpu_sc as plsc

_DIM = 32
_VOCAB = 1000000
_BATCH = 16384
_HIST = 50
_NC = 2
_NS = 16
_NW = _NC * _NS
_BW = _BATCH // _NW
_TC = _BW // 128
_NBLKF = _VOCAB // 128
_BLK_PER_W = _NBLKF // _NW
_TAIL0 = _BLK_PER_W * _NW


def _make_relayout():
    mesh = plsc.VectorSubcoreMesh(core_axis_name="c", subcore_axis_name="s")

    @functools.partial(
        pl.kernel,
        mesh=mesh,
        out_type=jax.ShapeDtypeStruct((_VOCAB * _DIM // 128, 128), jnp.float32),
        scratch_types=[
            pltpu.VMEM((_DIM, 128), jnp.float32),
            pltpu.VMEM((_DIM, 128), jnp.float32),
            pltpu.VMEM((_DIM, 128), jnp.float32),
            pltpu.VMEM((_DIM, 128), jnp.float32),
            pltpu.SemaphoreType.DMA,
            pltpu.SemaphoreType.DMA,
            pltpu.SemaphoreType.DMA,
            pltpu.SemaphoreType.DMA,
        ],
        compiler_params=pltpu.CompilerParams(use_tc_tiling_on_sc=True,
                                             needs_layout_passes=False),
    )
    def relayout_kernel(embt_hbm, tail_hbm, out_hbm,
                        blk_v0, blk_v1, stag_v0, stag_v1,
                        sem_i0, sem_i1, sem_o0, sem_o1):
        blk_v = (blk_v0, blk_v1)
        stag_v = (stag_v0, stag_v1)
        sem_i = (sem_i0, sem_i1)
        sem_o = (sem_o0, sem_o1)

        wid = lax.axis_index("s") * _NC + lax.axis_index("c")
        iota = lax.iota(jnp.int32, 16)

        def v0_of(t):
            return pl.multiple_of((wid + t * _NW) * 128, 128)

        def in_copy(t, b):
            return pltpu.make_async_copy(
                embt_hbm.at[:, pl.ds(v0_of(t), 128)], blk_v[b], sem_i[b])

        def out_copy(t, b):
            return pltpu.make_async_copy(
                stag_v[b], out_hbm.at[pl.ds(pl.multiple_of(v0_of(t) // 4, 32),
                                            _DIM)], sem_o[b])

        iota32 = iota * _DIM

        def transpose(b):
            blk = blk_v[b]
            stag = stag_v[b]

            def kbody(k, carry):
                dl = lax.bitwise_and(iota + k, 15)
                for d0 in (0, 16):
                    for sb in range(8):
                        vals = plsc.load_gather(blk, [d0 + dl, sb * 16 + iota])
                        addr = dl + (iota32 + (sb * 512 + d0))
                        plsc.store_scatter(
                            stag,
                            [lax.shift_right_logical(addr, 7),
                             lax.bitwise_and(addr, 127)],
                            vals)
                return carry

            lax.fori_loop(0, 16, kbody, 0)

        in_copy(0, 0).start()
        in_copy(1, 1).start()

        def step(t, b):
            in_copy(t, b).wait()

            @pl.when(t >= 2)
            def _():
                out_copy(t - 2, b).wait()

            transpose(b)
            out_copy(t, b).start()

            @pl.when(t + 2 < _BLK_PER_W)
            def _():
                in_copy(t + 2, b).start()

        def pair(i, carry):
            step(2 * i, 0)
            step(2 * i + 1, 1)
            return carry

        lax.fori_loop(0, _BLK_PER_W // 2, pair, 0)
        out_copy(_BLK_PER_W - 2, 0).wait()
        out_copy(_BLK_PER_W - 1, 1).wait()

        @pl.when(wid < 4)
        def _():
            v0 = pl.multiple_of((_TAIL0 + wid) * 128, 128)
            pltpu.sync_copy(embt_hbm.at[:, pl.ds(v0, 128)], blk_v0)
            transpose(0)
            pltpu.sync_copy(
                stag_v0,
                out_hbm.at[pl.ds(pl.multiple_of(v0 // 4, 32), _DIM)])

        @pl.when(wid == 4)
        def _():
            pltpu.sync_copy(tail_hbm, stag_v0.at[pl.ds(0, 16)])
            pltpu.sync_copy(stag_v0.at[pl.ds(0, 16)],
                            out_hbm.at[pl.ds(_NBLKF * 128 // 4, 16)])

    return relayout_kernel


def _make_gather():
    mesh = plsc.VectorSubcoreMesh(core_axis_name="c", subcore_axis_name="s")

    @functools.partial(
        pl.kernel,
        mesh=mesh,
        out_type=jax.ShapeDtypeStruct((_HIST, _DIM // 8, _BATCH // 128, 1024),
                                      jnp.float32),
        scratch_types=[
            pltpu.VMEM((_BW,), jnp.int32),
            pltpu.VMEM((_BW,), jnp.int32),
            pltpu.VMEM((_BW, _DIM), jnp.float32),
            pltpu.VMEM((_BW, _DIM), jnp.float32),
            pltpu.VMEM((_DIM // 8, _TC, 1024), jnp.float32),
            pltpu.VMEM((_DIM // 8, _TC, 1024), jnp.float32),
            pltpu.SemaphoreType.DMA,
            pltpu.SemaphoreType.DMA,
            pltpu.SemaphoreType.DMA,
            pltpu.SemaphoreType.DMA,
            pltpu.SemaphoreType.DMA,
            pltpu.SemaphoreType.DMA,
        ],
        compiler_params=pltpu.CompilerParams(use_tc_tiling_on_sc=False,
                                             needs_layout_passes=False),
    )
    def gather_kernel(idx_hbm, table_hbm, out_hbm,
                      idx_v0, idx_v1, rows_v0, rows_v1, stag_v0, stag_v1,
                      sem_i0, sem_i1, sem_g0, sem_g1, sem_s0, sem_s1):
        idx_v = (idx_v0, idx_v1)
        rows_v = (rows_v0, rows_v1)
        stag_v = (stag_v0, stag_v1)
        sem_i = (sem_i0, sem_i1)
        sem_g = (sem_g0, sem_g1)
        sem_s = (sem_s0, sem_s1)

        wid = lax.axis_index("s") * _NC + lax.axis_index("c")
        b0 = wid * _BW
        tc0 = wid * _TC
        iota = lax.iota(jnp.int32, 16)

        def idx_copy(h, b):
            return pltpu.make_async_copy(
                idx_hbm.at[h, pl.ds(b0, _BW)], idx_v[b], sem_i[b])

        def gather_copy(b):
            return pltpu.make_async_copy(
                table_hbm.at[idx_v[b]], rows_v[b], sem_g[b])

        def store_copy(h, b):
            return pltpu.make_async_copy(
                stag_v[b], out_hbm.at[h, :, pl.ds(tc0, _TC)], sem_s[b])

        def transpose(b):
            rows = rows_v[b]
            stag = stag_v[b]

            def kbody(k, carry):
                dl = lax.bitwise_and(iota + k, 15)
                for d0 in (0, 16):
                    d_vec = d0 + dl
                    dt_vec = lax.shift_right_logical(d_vec, 3)
                    off0 = lax.bitwise_and(d_vec, 7) * 128 + iota
                    for bg in range(32):
                        vals = plsc.load_gather(rows, [bg * 16 + iota, d_vec])
                        plsc.store_scatter(
                            stag,
                            [dt_vec, iota * 0 + (bg >> 3),
                             off0 + (bg & 7) * 16],
                            vals)
                return carry

            lax.fori_loop(0, 16, kbody, 0)

        idx_copy(0, 0).start()
        idx_copy(1, 1).start()
        idx_copy(0, 0).wait()
        gather_copy(0).start()

        def step(h, b):
            gather_copy(b).wait()

            @pl.when(h + 2 < _HIST)
            def _():
                idx_copy(h + 2, b).start()

            @pl.when(h + 1 < _HIST)
            def _():
                idx_copy(h + 1, 1 - b).wait()
                gather_copy(1 - b).start()

            @pl.when(h >= 2)
            def _():
                store_copy(h, b).wait()

            transpose(b)
            store_copy(h, b).start()

        def pair(i, carry):
            step(2 * i, 0)
            step(2 * i + 1, 1)
            return carry

        lax.fori_loop(0, _HIST // 2, pair, 0)
        store_copy(_HIST - 2, 0).wait()
        store_copy(_HIST - 1, 1).wait()

    return gather_kernel


_relayout = _make_relayout()
_gather = _make_gather()


def kernel(x, embedding_matrix):
    xt = x.T.astype(jnp.int32)
    embt = embedding_matrix.T
    tail = jax.lax.slice(embedding_matrix, (_NBLKF * 128, 0),
                         (_VOCAB, _DIM)).reshape(16, 128)
    table = _relayout(embt, tail)
    table = table.reshape(_VOCAB, _DIM)
    out_phys = _gather(xt, table)
    out5 = out_phys.reshape(_HIST, _DIM // 8, _BATCH // 128, 8, 128)
    t = jnp.transpose(out5, (2, 4, 0, 1, 3))
    return t.reshape(_BATCH, _HIST, _DIM)


# speedup vs baseline: 1.0041x; 1.0041x over previous
import functools

import jax
import jax.numpy as jnp
from jax import lax
from jax.experimental import pallas as pl
from jax.experimental.pallas import tpu as pltpu
from jax.experimental.pallas import tpu_sc as plsc

_DIM = 32
_VOCAB = 1000000
_BATCH = 16384
_HIST = 50
_NC = 2
_NS = 16
_NW = _NC * _NS
_BW = _BATCH // _NW
_TC = _BW // 128
_NBLKF = _VOCAB // 128
_BLK_PER_W = _NBLKF // _NW
_TAIL0 = _BLK_PER_W * _NW


def _make_relayout():
    mesh = plsc.VectorSubcoreMesh(core_axis_name="c", subcore_axis_name="s")

    @functools.partial(
        pl.kernel,
        mesh=mesh,
        out_type=jax.ShapeDtypeStruct((_VOCAB * _DIM // 128, 128), jnp.float32),
        scratch_types=(
            [pltpu.VMEM((_DIM, 128), jnp.float32) for _ in range(8)]
            + [pltpu.SemaphoreType.DMA for _ in range(8)]
        ),
        compiler_params=pltpu.CompilerParams(use_tc_tiling_on_sc=True,
                                             needs_layout_passes=False),
    )
    def relayout_kernel(embt_hbm, tail_hbm, out_hbm, *sc):
        blk_v = sc[0:4]
        stag_v = sc[4:8]
        sem_i = sc[8:12]
        sem_o = sc[12:16]
        blk_v0 = blk_v[0]
        stag_v0 = stag_v[0]

        wid = lax.axis_index("s") * _NC + lax.axis_index("c")
        iota = lax.iota(jnp.int32, 16)

        def v0_of(t):
            return pl.multiple_of((wid + t * _NW) * 128, 128)

        def in_copy(t, b):
            return pltpu.make_async_copy(
                embt_hbm.at[:, pl.ds(v0_of(t), 128)], blk_v[b], sem_i[b])

        def out_copy(t, b):
            return pltpu.make_async_copy(
                stag_v[b], out_hbm.at[pl.ds(pl.multiple_of(v0_of(t) // 4, 32),
                                            _DIM)], sem_o[b])

        iota32 = iota * _DIM

        def transpose(b):
            blk = blk_v[b]
            stag = stag_v[b]

            def kbody(k, carry):
                dl = lax.bitwise_and(iota + k, 15)
                for d0 in (0, 16):
                    for sb in range(8):
                        vals = plsc.load_gather(blk, [d0 + dl, sb * 16 + iota])
                        addr = dl + (iota32 + (sb * 512 + d0))
                        plsc.store_scatter(
                            stag,
                            [lax.shift_right_logical(addr, 7),
                             lax.bitwise_and(addr, 127)],
                            vals)
                return carry

            lax.fori_loop(0, 16, kbody, 0)

        for p in range(4):
            in_copy(p, p).start()

        def step(t, b):
            in_copy(t, b).wait()

            @pl.when(t >= 4)
            def _():
                out_copy(t - 4, b).wait()

            transpose(b)
            out_copy(t, b).start()

            @pl.when(t + 4 < _BLK_PER_W)
            def _():
                in_copy(t + 4, b).start()

        def quad(i, carry):
            for p in range(4):
                step(4 * i + p, p)
            return carry

        lax.fori_loop(0, _BLK_PER_W // 4, quad, 0)
        for p in range(4):
            out_copy(_BLK_PER_W - 4 + p, p).wait()

        @pl.when(wid < 4)
        def _():
            v0 = pl.multiple_of((_TAIL0 + wid) * 128, 128)
            pltpu.sync_copy(embt_hbm.at[:, pl.ds(v0, 128)], blk_v0)
            transpose(0)
            pltpu.sync_copy(
                stag_v0,
                out_hbm.at[pl.ds(pl.multiple_of(v0 // 4, 32), _DIM)])

        @pl.when(wid == 4)
        def _():
            pltpu.sync_copy(tail_hbm, stag_v0.at[pl.ds(0, 16)])
            pltpu.sync_copy(stag_v0.at[pl.ds(0, 16)],
                            out_hbm.at[pl.ds(_NBLKF * 128 // 4, 16)])

    return relayout_kernel


def _make_gather():
    mesh = plsc.VectorSubcoreMesh(core_axis_name="c", subcore_axis_name="s")

    @functools.partial(
        pl.kernel,
        mesh=mesh,
        out_type=jax.ShapeDtypeStruct((_HIST, _DIM // 8, _BATCH // 128, 1024),
                                      jnp.float32),
        scratch_types=[
            pltpu.VMEM((_BW,), jnp.int32),
            pltpu.VMEM((_BW,), jnp.int32),
            pltpu.VMEM((_BW, _DIM), jnp.float32),
            pltpu.VMEM((_BW, _DIM), jnp.float32),
            pltpu.VMEM((_DIM // 8, _TC, 1024), jnp.float32),
            pltpu.VMEM((_DIM // 8, _TC, 1024), jnp.float32),
            pltpu.SemaphoreType.DMA,
            pltpu.SemaphoreType.DMA,
            pltpu.SemaphoreType.DMA,
            pltpu.SemaphoreType.DMA,
            pltpu.SemaphoreType.DMA,
            pltpu.SemaphoreType.DMA,
        ],
        compiler_params=pltpu.CompilerParams(use_tc_tiling_on_sc=False,
                                             needs_layout_passes=False),
    )
    def gather_kernel(idx_hbm, table_hbm, out_hbm,
                      idx_v0, idx_v1, rows_v0, rows_v1, stag_v0, stag_v1,
                      sem_i0, sem_i1, sem_g0, sem_g1, sem_s0, sem_s1):
        idx_v = (idx_v0, idx_v1)
        rows_v = (rows_v0, rows_v1)
        stag_v = (stag_v0, stag_v1)
        sem_i = (sem_i0, sem_i1)
        sem_g = (sem_g0, sem_g1)
        sem_s = (sem_s0, sem_s1)

        wid = lax.axis_index("s") * _NC + lax.axis_index("c")
        b0 = wid * _BW
        tc0 = wid * _TC
        iota = lax.iota(jnp.int32, 16)

        def idx_copy(h, b):
            return pltpu.make_async_copy(
                idx_hbm.at[h, pl.ds(b0, _BW)], idx_v[b], sem_i[b])

        def gather_copy(b):
            return pltpu.make_async_copy(
                table_hbm.at[idx_v[b]], rows_v[b], sem_g[b])

        def store_copy(h, b):
            return pltpu.make_async_copy(
                stag_v[b], out_hbm.at[h, :, pl.ds(tc0, _TC)], sem_s[b])

        def transpose(b):
            rows = rows_v[b]
            stag = stag_v[b]

            def kbody(k, carry):
                dl = lax.bitwise_and(iota + k, 15)
                for d0 in (0, 16):
                    d_vec = d0 + dl
                    dt_vec = lax.shift_right_logical(d_vec, 3)
                    off0 = lax.bitwise_and(d_vec, 7) * 128 + iota
                    for bg in range(32):
                        vals = plsc.load_gather(rows, [bg * 16 + iota, d_vec])
                        plsc.store_scatter(
                            stag,
                            [dt_vec, iota * 0 + (bg >> 3),
                             off0 + (bg & 7) * 16],
                            vals)
                return carry

            lax.fori_loop(0, 16, kbody, 0)

        idx_copy(0, 0).start()
        idx_copy(1, 1).start()
        idx_copy(0, 0).wait()
        gather_copy(0).start()

        def step(h, b):
            gather_copy(b).wait()

            @pl.when(h + 2 < _HIST)
            def _():
                idx_copy(h + 2, b).start()

            @pl.when(h + 1 < _HIST)
            def _():
                idx_copy(h + 1, 1 - b).wait()
                gather_copy(1 - b).start()

            @pl.when(h >= 2)
            def _():
                store_copy(h, b).wait()

            transpose(b)
            store_copy(h, b).start()

        def pair(i, carry):
            step(2 * i, 0)
            step(2 * i + 1, 1)
            return carry

        lax.fori_loop(0, _HIST // 2, pair, 0)
        store_copy(_HIST - 2, 0).wait()
        store_copy(_HIST - 1, 1).wait()

    return gather_kernel


_relayout = _make_relayout()
_gather = _make_gather()


def kernel(x, embedding_matrix):
    xt = x.T.astype(jnp.int32)
    embt = embedding_matrix.T
    tail = jax.lax.slice(embedding_matrix, (_NBLKF * 128, 0),
                         (_VOCAB, _DIM)).reshape(16, 128)
    table = _relayout(embt, tail)
    table = table.reshape(_VOCAB, _DIM)
    out_phys = _gather(xt, table)
    out5 = out_phys.reshape(_HIST, _DIM // 8, _BATCH // 128, 8, 128)
    t = jnp.transpose(out5, (2, 4, 0, 1, 3))
    return t.reshape(_BATCH, _HIST, _DIM)
